# baseline (speedup 1.0000x reference)
import jax
import jax.numpy as jnp
from jax import lax
from jax.experimental import pallas as pl
from jax.experimental.pallas import tpu as pltpu

_B = 1024
_M = 256
_D = 128
_NBUF = 16
_PRE = 8
_QOUT = 7

_SIZES = [4, 4, 8, 8, 16, 16] + [24] * 39 + [16, 8, 4, 2, 2]
_STARTS = [sum(_SIZES[:i]) for i in range(len(_SIZES))]
_NCHUNK = len(_SIZES)
_CBMAX = max(_SIZES)


def _body(state_sref, state_ref, z_ref, mem_ref, out_ref, ctr_ref,
          bufs, sem_in, sem_out):
    ctr_ref[...] = state_ref[...] + 1

    def cp_in(j, s):
        nb = _SIZES[j]
        return pltpu.make_async_copy(
            mem_ref.at[pl.ds(_STARTS[j] * _M, nb * _M)],
            bufs.at[s].at[pl.ds(0, nb * _M)], sem_in.at[s])

    def cp_out(j, s):
        nb = _SIZES[j]
        return pltpu.make_async_copy(
            bufs.at[s].at[pl.ds(0, nb * _M)],
            out_ref.at[pl.ds(_STARTS[j] * _M, nb * _M)], sem_out.at[s])

    def patch(j, s):
        for b in range(_SIZES[j]):
            gb = _STARTS[j] + b
            r = lax.rem(state_sref[gb], _M)
            bufs[s, pl.ds(b * _M + r, 1), :] = z_ref[pl.ds(gb, 1), :]

    for c in range(_PRE):
        cp_in(c, c % _NBUF).start()

    for j in range(_NCHUNK):
        s = j % _NBUF
        cp_in(j, s).wait()
        patch(j, s)
        cp_out(j, s).start()
        if j >= _QOUT:
            jq = j - _QOUT
            cp_out(jq, jq % _NBUF).wait()
        if j + _PRE < _NCHUNK:
            jn = j + _PRE
            cp_in(jn, jn % _NBUF).start()

    for q in range(_QOUT):
        j = _NCHUNK - _QOUT + q
        cp_out(j, j % _NBUF).wait()


def kernel(z, mem_state, state):
    b, m, d = mem_state.shape
    mem2d = mem_state.reshape(b * m, d)
    state2d = state.reshape(b, 1)
    grid_spec = pltpu.PrefetchScalarGridSpec(
        num_scalar_prefetch=1,
        grid=(1,),
        in_specs=[
            pl.BlockSpec((b, 1), lambda i, s_ref: (0, 0)),
            pl.BlockSpec((b, d), lambda i, s_ref: (0, 0)),
            pl.BlockSpec(memory_space=pltpu.MemorySpace.HBM),
        ],
        out_specs=[
            pl.BlockSpec(memory_space=pltpu.MemorySpace.HBM),
            pl.BlockSpec((b, 1), lambda i, s_ref: (0, 0)),
        ],
        scratch_shapes=[
            pltpu.VMEM((_NBUF, _CBMAX * _M, _D), jnp.float32),
            pltpu.SemaphoreType.DMA((_NBUF,)),
            pltpu.SemaphoreType.DMA((_NBUF,)),
        ],
    )
    out2d, ctr2d = pl.pallas_call(
        _body,
        grid_spec=grid_spec,
        out_shape=[
            jax.ShapeDtypeStruct((b * m, d), mem_state.dtype),
            jax.ShapeDtypeStruct((b, 1), state.dtype),
        ],
    )(state, state2d, z, mem2d)
    return out2d.reshape(b, m, d), ctr2d.reshape(b)
